# baseline (speedup 1.0000x reference)
import dataclasses
import functools

import jax
import jax.numpy as jnp
from jax import lax
from jax.experimental import pallas as pl
from jax.experimental.pallas import tpu as pltpu
from jax.experimental.pallas import tpu_sc as plsc

D = 128
B = 16384
NC = 2
NS = 16
L = 16
NW = NC * NS
BPW = B // NW
BLK = 128
NBLK = BPW // BLK

_mesh = plsc.VectorSubcoreMesh(core_axis_name="c", subcore_axis_name="s")

_sc_params = pltpu.CompilerParams()
if "needs_layout_passes" in pltpu.CompilerParams.__dataclass_fields__:
    _sc_params = dataclasses.replace(_sc_params, needs_layout_passes=False)


def _sc_inner(parent_idx, child_idx, embeddings):
    @functools.partial(
        pl.kernel,
        out_type=jax.ShapeDtypeStruct((B,), jnp.float32),
        mesh=_mesh,
        compiler_params=_sc_params,
        scratch_types=[
            pltpu.VMEM((BPW,), jnp.int32),
            pltpu.VMEM((BPW,), jnp.int32),
            pltpu.VMEM((2, BLK, D), jnp.float32),
            pltpu.VMEM((2, BLK, D), jnp.float32),
            pltpu.VMEM((BPW,), jnp.float32),
            pltpu.SemaphoreType.DMA,
            pltpu.SemaphoreType.DMA,
            pltpu.SemaphoreType.DMA,
            pltpu.SemaphoreType.DMA,
        ],
    )
    def k(pidx_hbm, cidx_hbm, table_hbm, out_hbm,
          pidx_v, cidx_v, prow_v, crow_v, out_v, psem0, psem1, csem0, csem1):
        wid = lax.axis_index("s") * NC + lax.axis_index("c")
        base = wid * BPW
        pltpu.sync_copy(pidx_hbm.at[pl.ds(base, BPW)], pidx_v)
        pltpu.sync_copy(cidx_hbm.at[pl.ds(base, BPW)], cidx_v)

        lanes = lax.iota(jnp.int32, L)
        sgn = jnp.where(lanes == 0, jnp.float32(-1.0), jnp.float32(1.0))

        psems = (psem0, psem1)
        csems = (csem0, csem1)
        handles = {}

        def start(blk):
            b = blk % 2
            handles[blk] = (
                pltpu.async_copy(
                    table_hbm.at[pidx_v.at[pl.ds(blk * BLK, BLK)]],
                    prow_v.at[b], psems[b]),
                pltpu.async_copy(
                    table_hbm.at[cidx_v.at[pl.ds(blk * BLK, BLK)]],
                    crow_v.at[b], csems[b]),
            )

        start(0)
        for blk in range(NBLK):
            if blk + 1 < NBLK:
                start(blk + 1)
            hp, hc = handles.pop(blk)
            hp.wait()
            hc.wait()
            b = blk % 2
            pb = prow_v.at[b]
            cb = crow_v.at[b]

            @plsc.parallel_loop(0, BLK // L, unroll=4)
            def group(g):
                res = jnp.zeros((L,), jnp.float32)
                for j in range(L):
                    w = g * L + j
                    acc = (pb[w, pl.ds(0, L)] * sgn) * cb[w, pl.ds(0, L)]
                    for kk in range(1, D // L):
                        acc = acc + (pb[w, pl.ds(kk * L, L)]
                                     * cb[w, pl.ds(kk * L, L)])
                    s = jnp.sum(acc)
                    res = jnp.where(lanes == j, s, res)
                out_v[pl.ds(blk * BLK + g * L, L)] = res

        pltpu.sync_copy(out_v, out_hbm.at[pl.ds(base, BPW)])

    return k(parent_idx, child_idx, embeddings)


def _tc_dist(inner, curvature_k):
    x2 = inner.reshape(B // D, D)
    kv = curvature_k.reshape(1)

    def body(k_ref, x_ref, o_ref):
        kk = k_ref[0]
        arg = jnp.maximum(-x_ref[...] / kk, jnp.float32(1.0 + 1e-7))
        acosh = jnp.log(arg + jnp.sqrt((arg - 1.0) * (arg + 1.0)))
        o_ref[...] = jnp.sqrt(kk) * acosh

    out = pl.pallas_call(
        body,
        out_shape=jax.ShapeDtypeStruct((B // D, D), jnp.float32),
        in_specs=[
            pl.BlockSpec(memory_space=pltpu.SMEM),
            pl.BlockSpec(memory_space=pltpu.VMEM),
        ],
        out_specs=pl.BlockSpec(memory_space=pltpu.VMEM),
    )(kv, x2)
    return out.reshape(B)


def kernel(parent_idx, child_idx, embeddings, curvature_k):
    pidx = parent_idx.astype(jnp.int32)
    cidx = child_idx.astype(jnp.int32)
    inner = _sc_inner(pidx, cidx, embeddings)
    return _tc_dist(inner, curvature_k)


# speedup vs baseline: 1.9204x; 1.9204x over previous
import dataclasses
import functools

import jax
import jax.numpy as jnp
from jax import lax
from jax.experimental import pallas as pl
from jax.experimental.pallas import tpu as pltpu
from jax.experimental.pallas import tpu_sc as plsc

D = 128
B = 16384
NC = 2
NS = 16
L = 16
NW = NC * NS
BPW = B // NW
BLK = 64
NBLK = BPW // BLK
NBUF = 4

_mesh = plsc.VectorSubcoreMesh(core_axis_name="c", subcore_axis_name="s")

_sc_params = pltpu.CompilerParams()
if "needs_layout_passes" in pltpu.CompilerParams.__dataclass_fields__:
    _sc_params = dataclasses.replace(_sc_params, needs_layout_passes=False)


def _sc_inner(parent_idx, child_idx, embeddings):
    @functools.partial(
        pl.kernel,
        out_type=jax.ShapeDtypeStruct((L, B), jnp.float32),
        mesh=_mesh,
        compiler_params=_sc_params,
        scratch_types=[
            pltpu.VMEM((BPW,), jnp.int32),
            pltpu.VMEM((BPW,), jnp.int32),
            pltpu.VMEM((NBUF, BLK, D), jnp.float32),
            pltpu.VMEM((NBUF, BLK, D), jnp.float32),
            pltpu.VMEM((L, BPW + 1), jnp.float32),
        ] + [pltpu.SemaphoreType.DMA] * (2 * NBUF + 2),
    )
    def k(pidx_hbm, cidx_hbm, table_hbm, out_hbm,
          pidx_v, cidx_v, prow_v, crow_v, out_v, *sems):
        wid = lax.axis_index("s") * NC + lax.axis_index("c")
        base = wid * BPW
        hi_p = pltpu.async_copy(pidx_hbm.at[pl.ds(base, BPW)], pidx_v, sems[-2])
        hi_c = pltpu.async_copy(cidx_hbm.at[pl.ds(base, BPW)], cidx_v, sems[-1])
        hi_p.wait()
        hi_c.wait()

        lanes = lax.iota(jnp.int32, L)
        sgn = jnp.where(lanes == 0, jnp.float32(-1.0), jnp.float32(1.0))

        psems = sems[:NBUF]
        csems = sems[NBUF:]
        handles = {}

        def start(blk):
            b = blk % NBUF
            handles[blk] = (
                pltpu.async_copy(
                    table_hbm.at[pidx_v.at[pl.ds(blk * BLK, BLK)]],
                    prow_v.at[b], psems[b]),
                pltpu.async_copy(
                    table_hbm.at[cidx_v.at[pl.ds(blk * BLK, BLK)]],
                    crow_v.at[b], csems[b]),
            )

        for blk in range(NBUF - 1):
            start(blk)
        for blk in range(NBLK):
            if blk + NBUF - 1 < NBLK:
                start(blk + NBUF - 1)
            hp, hc = handles.pop(blk)
            hp.wait()
            hc.wait()
            b = blk % NBUF
            pb = prow_v.at[b]
            cb = crow_v.at[b]

            @plsc.parallel_loop(0, BLK, unroll=1)
            def pair(w):
                acc = (pb[w, pl.ds(0, L)] * sgn) * cb[w, pl.ds(0, L)]
                for kk in range(1, D // L):
                    acc = acc + (pb[w, pl.ds(kk * L, L)]
                                 * cb[w, pl.ds(kk * L, L)])
                cols = jnp.full((L,), blk * BLK + w, jnp.int32)
                plsc.store_scatter(out_v, [lanes, cols], acc)

        pltpu.sync_copy(out_v.at[:, pl.ds(0, BPW)],
                        out_hbm.at[:, pl.ds(base, BPW)])

    return k(parent_idx, child_idx, embeddings)


def _tc_dist(partials, curvature_k):
    kv = curvature_k.reshape(1)

    def body(k_ref, x_ref, o_ref):
        kk = k_ref[0]
        inner = jnp.sum(x_ref[...], axis=0)
        arg = jnp.maximum(-inner / kk, jnp.float32(1.0 + 1e-7))
        acosh = jnp.log(arg + jnp.sqrt((arg - 1.0) * (arg + 1.0)))
        o_ref[...] = jnp.sqrt(kk) * acosh

    return pl.pallas_call(
        body,
        out_shape=jax.ShapeDtypeStruct((B,), jnp.float32),
        in_specs=[
            pl.BlockSpec(memory_space=pltpu.SMEM),
            pl.BlockSpec(memory_space=pltpu.VMEM),
        ],
        out_specs=pl.BlockSpec(memory_space=pltpu.VMEM),
    )(kv, partials)


def kernel(parent_idx, child_idx, embeddings, curvature_k):
    pidx = parent_idx.astype(jnp.int32)
    cidx = child_idx.astype(jnp.int32)
    partials = _sc_inner(pidx, cidx, embeddings)
    return _tc_dist(partials, curvature_k)
